# Initial kernel scaffold; baseline (speedup 1.0000x reference)
import jax
import jax.numpy as jnp
from jax.experimental import pallas as pl


def kernel(x, edge_index, W1, b1, W2, b2, W3, b3, fcW, fcb, fc2W, fc2b):
    raise NotImplementedError("write your pallas kernel here")


# speedup vs baseline: 2.7703x; 2.7703x over previous
import functools

import jax
import jax.numpy as jnp
from jax.experimental import pallas as pl
from jax.experimental.pallas import tpu as pltpu

N = 99990
E = 6399360
G = N // 30


def _head_body(h_ref, fcW_ref, fcb_ref, fc2W_ref, fc2b_ref, o_ref):
    h = h_ref[...]
    t = jnp.dot(h, fcW_ref[...], preferred_element_type=jnp.float32) + fcb_ref[...]
    o_ref[...] = (
        jnp.dot(t, fc2W_ref[...], preferred_element_type=jnp.float32) + fc2b_ref[...]
    )


def _head(hr, fcW, fcb, fc2W, fc2b):
    BG = 512
    grid = (pl.cdiv(G, BG),)
    return pl.pallas_call(
        _head_body,
        grid=grid,
        in_specs=[
            pl.BlockSpec((BG, 360), lambda i: (i, 0)),
            pl.BlockSpec((360, 120), lambda i: (0, 0)),
            pl.BlockSpec((120,), lambda i: (0,)),
            pl.BlockSpec((120, 36), lambda i: (0, 0)),
            pl.BlockSpec((36,), lambda i: (0,)),
        ],
        out_specs=pl.BlockSpec((BG, 36), lambda i: (i, 0)),
        out_shape=jax.ShapeDtypeStruct((G, 36), jnp.float32),
    )(hr, fcW, fcb, fc2W, fc2b)


def _gcn(x, src, dst, dinv, W, b):
    y = dinv[:, None] * (x @ W)
    agg = jnp.zeros((N, W.shape[1]), jnp.float32).at[dst].add(y[src])
    return jax.nn.relu(dinv[:, None] * (agg + y) + b)


def kernel(x, edge_index, W1, b1, W2, b2, W3, b3, fcW, fcb, fc2W, fc2b):
    src, dst = edge_index[0], edge_index[1]
    deg = jnp.zeros((N,), jnp.float32).at[dst].add(1.0) + 1.0
    dinv = jax.lax.rsqrt(deg)
    h = _gcn(x, src, dst, dinv, W1, b1)
    h = _gcn(h, src, dst, dinv, W2, b2)
    h = _gcn(h, src, dst, dinv, W3, b3)
    hr = h.reshape(G, 360)
    return _head(hr, fcW, fcb, fc2W, fc2b)


# speedup vs baseline: 18.0467x; 6.5143x over previous
import functools

import jax
import jax.numpy as jnp
from jax import lax
from jax.experimental import pallas as pl
from jax.experimental.pallas import tpu as pltpu
from jax.experimental.pallas import tpu_sc as plsc

N = 99990
E = 6399360
G = N // 30
D = 16
CH = 128
C_TOT = E // CH
NC = 2
NS = 16
ROWS = 6400
N_PAD = NC * 0 + NS * ROWS

_MESH = plsc.VectorSubcoreMesh(core_axis_name="c", subcore_axis_name="s")
_SC_PARAMS = pltpu.CompilerParams(use_tc_tiling_on_sc=False)


def _tile_chunk_range(c, s):
    sc_lo = (c * C_TOT) // NC
    sc_hi = ((c + 1) * C_TOT) // NC
    cnt = sc_hi - sc_lo
    t_lo = sc_lo + (s * cnt) // NS
    t_hi = sc_lo + ((s + 1) * cnt) // NS
    return t_lo, t_hi


def _agg_body(ei_hbm, y_hbm, zeros_hbm, out_hbm, sidx, didx, rows, acc):
    c = lax.axis_index("c")
    s = lax.axis_index("s")
    lo = s * ROWS
    pltpu.sync_copy(zeros_hbm.at[pl.ds(lo, ROWS)], acc.at[pl.ds(lo, ROWS)])
    plsc.subcore_barrier()
    t_lo, t_hi = _tile_chunk_range(c, s)

    @pl.loop(t_lo, t_hi)
    def _(ci):
        base = ci * CH
        pltpu.sync_copy(ei_hbm.at[0, pl.ds(base, CH)], sidx)
        pltpu.sync_copy(ei_hbm.at[1, pl.ds(base, CH)], didx)
        pltpu.sync_copy(y_hbm.at[sidx], rows)
        pltpu.sync_copy(rows, acc.at[didx], add=True)

    plsc.subcore_barrier()
    pltpu.sync_copy(acc.at[pl.ds(lo, ROWS)], out_hbm.at[c, pl.ds(lo, ROWS)])


@functools.partial(
    pl.kernel,
    out_type=jax.ShapeDtypeStruct((NC, N_PAD, D), jnp.float32),
    mesh=_MESH,
    scratch_types=[
        pltpu.VMEM((CH,), jnp.int32),
        pltpu.VMEM((CH,), jnp.int32),
        pltpu.VMEM((CH, D), jnp.float32),
        pltpu.VMEM_SHARED((N_PAD, D), jnp.float32),
    ],
    compiler_params=_SC_PARAMS,
)
def _sc_agg(ei_hbm, y_hbm, zeros_hbm, out_hbm, sidx, didx, rows, acc):
    _agg_body(ei_hbm, y_hbm, zeros_hbm, out_hbm, sidx, didx, rows, acc)


def _deg_body(ei_hbm, zeros_hbm, out_hbm, didx, ones, acc):
    c = lax.axis_index("c")
    s = lax.axis_index("s")
    lo = s * ROWS
    pltpu.sync_copy(zeros_hbm.at[pl.ds(lo, ROWS)], acc.at[pl.ds(lo, ROWS)])

    @pl.loop(0, CH)
    def _(i):
        ones[i, :] = jnp.ones((D,), jnp.float32)

    plsc.subcore_barrier()
    t_lo, t_hi = _tile_chunk_range(c, s)

    @pl.loop(t_lo, t_hi)
    def _(ci):
        pltpu.sync_copy(ei_hbm.at[1, pl.ds(ci * CH, CH)], didx)
        pltpu.sync_copy(ones, acc.at[didx], add=True)

    plsc.subcore_barrier()
    pltpu.sync_copy(acc.at[pl.ds(lo, ROWS)], out_hbm.at[c, pl.ds(lo, ROWS)])


@functools.partial(
    pl.kernel,
    out_type=jax.ShapeDtypeStruct((NC, N_PAD, D), jnp.float32),
    mesh=_MESH,
    scratch_types=[
        pltpu.VMEM((CH,), jnp.int32),
        pltpu.VMEM((CH, D), jnp.float32),
        pltpu.VMEM_SHARED((N_PAD, D), jnp.float32),
    ],
    compiler_params=_SC_PARAMS,
)
def _sc_deg(ei_hbm, zeros_hbm, out_hbm, didx, ones, acc):
    _deg_body(ei_hbm, zeros_hbm, out_hbm, didx, ones, acc)


_BN = 2048


def _node_call(body, n_out, *args):
    grid = (pl.cdiv(N, _BN),)
    spec = pl.BlockSpec((_BN, D), lambda i: (i, 0))

    def mk_spec(a):
        if a.ndim == 1:
            return pl.BlockSpec(a.shape, lambda i: (0,))
        if a.shape[0] in (N, N_PAD):
            return pl.BlockSpec((_BN, a.shape[1]), lambda i: (i, 0))
        return pl.BlockSpec(a.shape, lambda i: (0,) * a.ndim)

    return pl.pallas_call(
        body,
        grid=grid,
        in_specs=[mk_spec(a) for a in args],
        out_specs=[spec] * n_out if n_out > 1 else spec,
        out_shape=(
            [jax.ShapeDtypeStruct((N, D), jnp.float32)] * n_out
            if n_out > 1
            else jax.ShapeDtypeStruct((N, D), jnp.float32)
        ),
    )(*args)


def _tc0_body(x_ref, dA_ref, dB_ref, w1_ref, dinv_ref, y1_ref):
    deg = dA_ref[...] + dB_ref[...] + 1.0
    dinv = lax.rsqrt(deg)
    dinv_ref[...] = dinv
    y1_ref[...] = dinv * jnp.dot(
        x_ref[...], w1_ref[...], preferred_element_type=jnp.float32
    )


def _tc_mid_body(dinv_ref, y_ref, aA_ref, aB_ref, b_ref, w_ref, yn_ref):
    dinv = dinv_ref[...]
    h = jax.nn.relu(dinv * (aA_ref[...] + aB_ref[...] + y_ref[...]) + b_ref[...])
    yn_ref[...] = dinv * jnp.dot(
        h, w_ref[...], preferred_element_type=jnp.float32
    )


def _tc2_body(
    dinv_ref, ya_ref, yb_ref, aAa_ref, aBa_ref, aAb_ref, aBb_ref,
    ba_ref, bb_ref, wa_ref, wb_ref, yn_ref,
):
    dinv = dinv_ref[...]
    ha = jax.nn.relu(dinv * (aAa_ref[...] + aBa_ref[...] + ya_ref[...]) + ba_ref[...])
    hb = jax.nn.relu(dinv * (aAb_ref[...] + aBb_ref[...] + yb_ref[...]) + bb_ref[...])
    yn_ref[...] = dinv * (
        jnp.dot(ha, wa_ref[...], preferred_element_type=jnp.float32)
        + jnp.dot(hb, wb_ref[...], preferred_element_type=jnp.float32)
    )


def _tc3_body(dinv_ref, y_ref, aA_ref, aB_ref, b_ref, h_ref):
    dinv = dinv_ref[...]
    h_ref[...] = jax.nn.relu(
        dinv * (aA_ref[...] + aB_ref[...] + y_ref[...]) + b_ref[...]
    )


def _head_body(h_ref, fcW_ref, fcb_ref, fc2W_ref, fc2b_ref, o_ref):
    t = jnp.dot(h_ref[...], fcW_ref[...], preferred_element_type=jnp.float32)
    t = t + fcb_ref[...]
    o_ref[...] = (
        jnp.dot(t, fc2W_ref[...], preferred_element_type=jnp.float32) + fc2b_ref[...]
    )


def _head(hr, fcW, fcb, fc2W, fc2b):
    BG = 512
    return pl.pallas_call(
        _head_body,
        grid=(pl.cdiv(G, BG),),
        in_specs=[
            pl.BlockSpec((BG, 360), lambda i: (i, 0)),
            pl.BlockSpec((360, 120), lambda i: (0, 0)),
            pl.BlockSpec((120,), lambda i: (0,)),
            pl.BlockSpec((120, 36), lambda i: (0, 0)),
            pl.BlockSpec((36,), lambda i: (0,)),
        ],
        out_specs=pl.BlockSpec((BG, 36), lambda i: (i, 0)),
        out_shape=jax.ShapeDtypeStruct((G, 36), jnp.float32),
    )(hr, fcW, fcb, fc2W, fc2b)


def _pad2(w, r, c):
    return jnp.pad(w, ((0, r - w.shape[0]), (0, c - w.shape[1])))


def kernel(x, edge_index, W1, b1, W2, b2, W3, b3, fcW, fcb, fc2W, fc2b):
    zeros2 = jnp.zeros((N_PAD, D), jnp.float32)
    W1p = _pad2(W1, 6, D)
    W2ap = _pad2(W2[:, :12], D, D)
    W2bp = _pad2(W2[:, 12:], D, D)
    W3ap = _pad2(W3[:12, :], D, D)
    W3bp = _pad2(W3[12:, :], D, D)
    b1p = jnp.pad(b1, (0, D - 12))
    b2a = jnp.pad(b2[:12], (0, D - 12))
    b2b = jnp.pad(b2[12:], (0, D - 12))
    b3p = jnp.pad(b3, (0, D - 12))

    deg = _sc_deg(edge_index, zeros2)
    dinv, y1 = _node_call(_tc0_body, 2, x, deg[0], deg[1], W1p)

    a1 = _sc_agg(edge_index, y1, zeros2)
    y2a = _node_call(_tc_mid_body, 1, dinv, y1, a1[0], a1[1], b1p, W2ap)
    y2b = _node_call(_tc_mid_body, 1, dinv, y1, a1[0], a1[1], b1p, W2bp)

    a2a = _sc_agg(edge_index, y2a, zeros2)
    a2b = _sc_agg(edge_index, y2b, zeros2)
    y3 = _node_call(
        _tc2_body, 1, dinv, y2a, y2b, a2a[0], a2a[1], a2b[0], a2b[1],
        b2a, b2b, W3ap, W3bp,
    )

    a3 = _sc_agg(edge_index, y3, zeros2)
    h3 = _node_call(_tc3_body, 1, dinv, y3, a3[0], a3[1], b3p)

    hr = h3[:, :12].reshape(G, 360)
    return _head(hr, fcW, fcb, fc2W, fc2b)


# speedup vs baseline: 43.9364x; 2.4346x over previous
import functools

import jax
import jax.numpy as jnp
from jax import lax
from jax.experimental import pallas as pl
from jax.experimental.pallas import tpu as pltpu
from jax.experimental.pallas import tpu_sc as plsc

N = 99990
E = 6399360
G = N // 30
D = 16
CH = 128
K = 4
SUP = K * CH
S_TOT = -(-E // SUP)
E_PAD = S_TOT * SUP
NC = 2
NS = 16
NW = NC * NS
ROWS = 6400
N_PAD = NS * ROWS

_MESH = plsc.VectorSubcoreMesh(core_axis_name="c", subcore_axis_name="s")
_SC_PARAMS = pltpu.CompilerParams(use_tc_tiling_on_sc=False)


def _tile_super_range(c, s):
    w = c * NS + s
    return (w * S_TOT) // NW, ((w + 1) * S_TOT) // NW


def _zero_acc(s, zeros_hbm, acc):
    lo = s * ROWS
    pltpu.sync_copy(zeros_hbm.at[pl.ds(lo, ROWS)], acc.at[pl.ds(lo, ROWS)])


def _writeback(c, s, acc, out_hbm):
    lo = s * ROWS
    pltpu.sync_copy(acc.at[pl.ds(lo, ROWS)], out_hbm.at[c, pl.ds(lo, ROWS)])


def _agg_body(ei_hbm, y_hbm, zeros_hbm, out_hbm, si, dis, rows, acc, isem, gsem, ssem):
    c = lax.axis_index("c")
    s = lax.axis_index("s")
    _zero_acc(s, zeros_hbm, acc)
    plsc.subcore_barrier()
    t_lo, t_hi = _tile_super_range(c, s)

    @pl.loop(t_lo, t_hi)
    def _(sup):
        i0 = pltpu.async_copy(ei_hbm.at[0, sup], si, isem)
        ii = [
            pltpu.async_copy(ei_hbm.at[1, sup, j], dis[j], isem) for j in range(K)
        ]
        i0.wait()
        for i1 in ii:
            i1.wait()
        gs = [
            pltpu.async_copy(
                y_hbm.at[si.at[j]], rows.at[pl.ds(j * CH, CH)], gsem
            )
            for j in range(K)
        ]
        for g in gs:
            g.wait()
        ss = [
            pltpu.async_copy(
                rows.at[pl.ds(j * CH, CH)], acc.at[dis[j]], ssem, add=True
            )
            for j in range(K)
        ]
        for t in ss:
            t.wait()

    plsc.subcore_barrier()
    _writeback(c, s, acc, out_hbm)


@functools.partial(
    pl.kernel,
    out_type=jax.ShapeDtypeStruct((NC, N_PAD, D), jnp.float32),
    mesh=_MESH,
    scratch_types=[
        pltpu.VMEM((K, CH), jnp.int32),
        [pltpu.VMEM((CH,), jnp.int32) for _ in range(K)],
        pltpu.VMEM((SUP, D), jnp.float32),
        pltpu.VMEM_SHARED((N_PAD, D), jnp.float32),
        pltpu.SemaphoreType.DMA,
        pltpu.SemaphoreType.DMA,
        pltpu.SemaphoreType.DMA,
    ],
    compiler_params=_SC_PARAMS,
)
def _sc_agg(ei_hbm, y_hbm, zeros_hbm, out_hbm, si, dis, rows, acc, isem, gsem, ssem):
    _agg_body(ei_hbm, y_hbm, zeros_hbm, out_hbm, si, dis, rows, acc, isem, gsem, ssem)


def _deg_body(ei_hbm, ones_hbm, zeros_hbm, out_hbm, dis, ones, acc, isem, ssem):
    c = lax.axis_index("c")
    s = lax.axis_index("s")
    _zero_acc(s, zeros_hbm, acc)
    pltpu.sync_copy(ones_hbm, ones)
    plsc.subcore_barrier()
    t_lo, t_hi = _tile_super_range(c, s)

    @pl.loop(t_lo, t_hi)
    def _(sup):
        ii = [
            pltpu.async_copy(ei_hbm.at[1, sup, j], dis[j], isem) for j in range(K)
        ]
        for i1 in ii:
            i1.wait()
        ss = [
            pltpu.async_copy(
                ones.at[pl.ds(j * CH, CH)], acc.at[dis[j]], ssem, add=True
            )
            for j in range(K)
        ]
        for t in ss:
            t.wait()

    plsc.subcore_barrier()
    _writeback(c, s, acc, out_hbm)


@functools.partial(
    pl.kernel,
    out_type=jax.ShapeDtypeStruct((NC, N_PAD, D), jnp.float32),
    mesh=_MESH,
    scratch_types=[
        [pltpu.VMEM((CH,), jnp.int32) for _ in range(K)],
        pltpu.VMEM((SUP, D), jnp.float32),
        pltpu.VMEM_SHARED((N_PAD, D), jnp.float32),
        pltpu.SemaphoreType.DMA,
        pltpu.SemaphoreType.DMA,
    ],
    compiler_params=_SC_PARAMS,
)
def _sc_deg(ei_hbm, ones_hbm, zeros_hbm, out_hbm, dis, ones, acc, isem, ssem):
    _deg_body(ei_hbm, ones_hbm, zeros_hbm, out_hbm, dis, ones, acc, isem, ssem)


_BN = 2048


def _node_call(body, n_out, *args):
    grid = (pl.cdiv(N, _BN),)
    spec = pl.BlockSpec((_BN, D), lambda i: (i, 0))

    def mk_spec(a):
        if a.ndim == 1:
            return pl.BlockSpec(a.shape, lambda i: (0,))
        if a.shape[0] in (N, N_PAD):
            return pl.BlockSpec((_BN, a.shape[1]), lambda i: (i, 0))
        return pl.BlockSpec(a.shape, lambda i: (0,) * a.ndim)

    return pl.pallas_call(
        body,
        grid=grid,
        in_specs=[mk_spec(a) for a in args],
        out_specs=[spec] * n_out if n_out > 1 else spec,
        out_shape=(
            [jax.ShapeDtypeStruct((N, D), jnp.float32)] * n_out
            if n_out > 1
            else jax.ShapeDtypeStruct((N, D), jnp.float32)
        ),
    )(*args)


def _tc0_body(x_ref, dA_ref, dB_ref, w1_ref, dinv_ref, y1_ref):
    deg = dA_ref[...] + dB_ref[...] + 1.0
    dinv = lax.rsqrt(deg)
    dinv_ref[...] = dinv
    y1_ref[...] = dinv * jnp.dot(
        x_ref[...], w1_ref[...], preferred_element_type=jnp.float32
    )


def _tc_mid_body(dinv_ref, y_ref, aA_ref, aB_ref, b_ref, w_ref, yn_ref):
    dinv = dinv_ref[...]
    h = jax.nn.relu(dinv * (aA_ref[...] + aB_ref[...] + y_ref[...]) + b_ref[...])
    yn_ref[...] = dinv * jnp.dot(
        h, w_ref[...], preferred_element_type=jnp.float32
    )


def _tc2_body(
    dinv_ref, ya_ref, yb_ref, aAa_ref, aBa_ref, aAb_ref, aBb_ref,
    ba_ref, bb_ref, wa_ref, wb_ref, yn_ref,
):
    dinv = dinv_ref[...]
    ha = jax.nn.relu(dinv * (aAa_ref[...] + aBa_ref[...] + ya_ref[...]) + ba_ref[...])
    hb = jax.nn.relu(dinv * (aAb_ref[...] + aBb_ref[...] + yb_ref[...]) + bb_ref[...])
    yn_ref[...] = dinv * (
        jnp.dot(ha, wa_ref[...], preferred_element_type=jnp.float32)
        + jnp.dot(hb, wb_ref[...], preferred_element_type=jnp.float32)
    )


def _tc3_body(dinv_ref, y_ref, aA_ref, aB_ref, b_ref, h_ref):
    dinv = dinv_ref[...]
    h_ref[...] = jax.nn.relu(
        dinv * (aA_ref[...] + aB_ref[...] + y_ref[...]) + b_ref[...]
    )


def _head_body(h_ref, fcW_ref, fcb_ref, fc2W_ref, fc2b_ref, o_ref):
    t = jnp.dot(h_ref[...], fcW_ref[...], preferred_element_type=jnp.float32)
    t = t + fcb_ref[...]
    o_ref[...] = (
        jnp.dot(t, fc2W_ref[...], preferred_element_type=jnp.float32) + fc2b_ref[...]
    )


def _head(hr, fcW, fcb, fc2W, fc2b):
    BG = 512
    return pl.pallas_call(
        _head_body,
        grid=(pl.cdiv(G, BG),),
        in_specs=[
            pl.BlockSpec((BG, 360), lambda i: (i, 0)),
            pl.BlockSpec((360, 120), lambda i: (0, 0)),
            pl.BlockSpec((120,), lambda i: (0,)),
            pl.BlockSpec((120, 36), lambda i: (0, 0)),
            pl.BlockSpec((36,), lambda i: (0,)),
        ],
        out_specs=pl.BlockSpec((BG, 36), lambda i: (i, 0)),
        out_shape=jax.ShapeDtypeStruct((G, 36), jnp.float32),
    )(hr, fcW, fcb, fc2W, fc2b)


def _pad2(w, r, c):
    return jnp.pad(w, ((0, r - w.shape[0]), (0, c - w.shape[1])))


def kernel(x, edge_index, W1, b1, W2, b2, W3, b3, fcW, fcb, fc2W, fc2b):
    zeros2 = jnp.zeros((N_PAD, D), jnp.float32)
    pad_n = E_PAD - E
    pad_edges = jnp.concatenate(
        [jnp.zeros((1, pad_n), jnp.int32), jnp.full((1, pad_n), N, jnp.int32)], axis=0
    )
    ei = jnp.concatenate([edge_index, pad_edges], axis=1).reshape(2, S_TOT, K, CH)
    W1p = _pad2(W1, 6, D)
    W2ap = _pad2(W2[:, :12], D, D)
    W2bp = _pad2(W2[:, 12:], D, D)
    W3ap = _pad2(W3[:12, :], D, D)
    W3bp = _pad2(W3[12:, :], D, D)
    b1p = jnp.pad(b1, (0, D - 12))
    b2a = jnp.pad(b2[:12], (0, D - 12))
    b2b = jnp.pad(b2[12:], (0, D - 12))
    b3p = jnp.pad(b3, (0, D - 12))

    ones2 = jnp.ones((SUP, D), jnp.float32)
    deg = _sc_deg(ei, ones2, zeros2)
    dinv, y1 = _node_call(_tc0_body, 2, x, deg[0], deg[1], W1p)

    a1 = _sc_agg(ei, y1, zeros2)
    y2a = _node_call(_tc_mid_body, 1, dinv, y1, a1[0], a1[1], b1p, W2ap)
    y2b = _node_call(_tc_mid_body, 1, dinv, y1, a1[0], a1[1], b1p, W2bp)

    a2a = _sc_agg(ei, y2a, zeros2)
    a2b = _sc_agg(ei, y2b, zeros2)
    y3 = _node_call(
        _tc2_body, 1, dinv, y2a, y2b, a2a[0], a2a[1], a2b[0], a2b[1],
        b2a, b2b, W3ap, W3bp,
    )

    a3 = _sc_agg(ei, y3, zeros2)
    h3 = _node_call(_tc3_body, 1, dinv, y3, a3[0], a3[1], b3p)

    hr = h3[:, :12].reshape(G, 360)
    return _head(hr, fcW, fcb, fc2W, fc2b)


# speedup vs baseline: 62.9824x; 1.4335x over previous
import functools

import jax
import jax.numpy as jnp
from jax import lax
from jax.experimental import pallas as pl
from jax.experimental.pallas import tpu as pltpu
from jax.experimental.pallas import tpu_sc as plsc

N = 99990
E = 6399360
G = N // 30
D = 16
CH = 128
K = 5
SUP = K * CH
S_TOT = E // SUP
NC = 2
NS = 16
NW = NC * NS
ROWS = 6400
N_PAD = NS * ROWS
RV = N_PAD * D // 128

_MESH = plsc.VectorSubcoreMesh(core_axis_name="c", subcore_axis_name="s")
_SC_PARAMS = pltpu.CompilerParams(use_tc_tiling_on_sc=False)


def _tile_super_range(c, s):
    w = c * NS + s
    return (w * S_TOT) // NW, ((w + 1) * S_TOT) // NW


def _zero_acc(s, zeros_hbm, acc):
    lo = s * ROWS
    pltpu.sync_copy(zeros_hbm.at[pl.ds(lo, ROWS)], acc.at[pl.ds(lo, ROWS)])


def _writeback(c, s, acc, out_hbm):
    lo = s * ROWS
    pltpu.sync_copy(acc.at[pl.ds(lo, ROWS)], out_hbm.at[c, pl.ds(lo, ROWS)])


def _agg_body(src_hbm, dst_hbm, y_hbm, zeros_hbm, out_hbm,
              si, dis, rows, acc, isem, gsem, ssem):
    c = lax.axis_index("c")
    s = lax.axis_index("s")
    _zero_acc(s, zeros_hbm, acc)
    plsc.subcore_barrier()
    t_lo, t_hi = _tile_super_range(c, s)

    @pl.loop(t_lo, t_hi)
    def _(sup):
        i0 = pltpu.async_copy(src_hbm.at[sup], si, isem)
        ii = [
            pltpu.async_copy(dst_hbm.at[sup, j], dis[j], isem) for j in range(K)
        ]
        i0.wait()
        for i1 in ii:
            i1.wait()
        gs = [
            pltpu.async_copy(
                y_hbm.at[si.at[j]], rows.at[pl.ds(j * CH, CH)], gsem
            )
            for j in range(K)
        ]
        ss = []
        for j in range(K):
            gs[j].wait()
            ss.append(
                pltpu.async_copy(
                    rows.at[pl.ds(j * CH, CH)], acc.at[dis[j]], ssem, add=True
                )
            )
        for t in ss:
            t.wait()

    plsc.subcore_barrier()
    _writeback(c, s, acc, out_hbm)


@functools.partial(
    pl.kernel,
    out_type=jax.ShapeDtypeStruct((NC, N_PAD, D), jnp.float32),
    mesh=_MESH,
    scratch_types=[
        pltpu.VMEM((K, CH), jnp.int32),
        [pltpu.VMEM((CH,), jnp.int32) for _ in range(K)],
        pltpu.VMEM((SUP, D), jnp.float32),
        pltpu.VMEM_SHARED((N_PAD, D), jnp.float32),
        pltpu.SemaphoreType.DMA,
        pltpu.SemaphoreType.DMA,
        pltpu.SemaphoreType.DMA,
    ],
    compiler_params=_SC_PARAMS,
)
def _sc_agg(src_hbm, dst_hbm, y_hbm, zeros_hbm, out_hbm,
            si, dis, rows, acc, isem, gsem, ssem):
    _agg_body(src_hbm, dst_hbm, y_hbm, zeros_hbm, out_hbm,
              si, dis, rows, acc, isem, gsem, ssem)


def _deg_body(dst_hbm, ones_hbm, zeros_hbm, out_hbm, dis, ones, acc, isem, ssem):
    c = lax.axis_index("c")
    s = lax.axis_index("s")
    _zero_acc(s, zeros_hbm, acc)
    pltpu.sync_copy(ones_hbm, ones)
    plsc.subcore_barrier()
    t_lo, t_hi = _tile_super_range(c, s)

    @pl.loop(t_lo, t_hi)
    def _(sup):
        ii = [
            pltpu.async_copy(dst_hbm.at[sup, j], dis[j], isem) for j in range(K)
        ]
        for i1 in ii:
            i1.wait()
        ss = [
            pltpu.async_copy(
                ones.at[pl.ds(j * CH, CH)], acc.at[dis[j]], ssem, add=True
            )
            for j in range(K)
        ]
        for t in ss:
            t.wait()

    plsc.subcore_barrier()
    _writeback(c, s, acc, out_hbm)


@functools.partial(
    pl.kernel,
    out_type=jax.ShapeDtypeStruct((NC, N_PAD, D), jnp.float32),
    mesh=_MESH,
    scratch_types=[
        [pltpu.VMEM((CH,), jnp.int32) for _ in range(K)],
        pltpu.VMEM((SUP, D), jnp.float32),
        pltpu.VMEM_SHARED((N_PAD, D), jnp.float32),
        pltpu.SemaphoreType.DMA,
        pltpu.SemaphoreType.DMA,
    ],
    compiler_params=_SC_PARAMS,
)
def _sc_deg(dst_hbm, ones_hbm, zeros_hbm, out_hbm, dis, ones, acc, isem, ssem):
    _deg_body(dst_hbm, ones_hbm, zeros_hbm, out_hbm, dis, ones, acc, isem, ssem)


_BV = 2560


def _node_call(body, n_out, *args):
    grid = (RV // _BV,)
    spec = pl.BlockSpec((_BV, 128), lambda i: (i, 0))

    def mk_spec(a):
        if a.ndim == 2 and a.shape[0] == RV:
            return pl.BlockSpec((_BV, 128), lambda i: (i, 0))
        return pl.BlockSpec(a.shape, lambda i: (0,) * a.ndim)

    return pl.pallas_call(
        body,
        grid=grid,
        in_specs=[mk_spec(a) for a in args],
        out_specs=[spec] * n_out if n_out > 1 else spec,
        out_shape=(
            [jax.ShapeDtypeStruct((RV, 128), jnp.float32)] * n_out
            if n_out > 1
            else jax.ShapeDtypeStruct((RV, 128), jnp.float32)
        ),
    )(*args)


def _tc0_body(x_ref, dA_ref, dB_ref, w1_ref, dinv_ref, y1_ref):
    deg = dA_ref[...] + dB_ref[...] + 1.0
    dinv = lax.rsqrt(deg)
    dinv_ref[...] = dinv
    y1_ref[...] = dinv * jnp.dot(
        x_ref[...], w1_ref[...], preferred_element_type=jnp.float32
    )


def _tc_mid_body(dinv_ref, y_ref, aA_ref, aB_ref, b_ref, w_ref, yn_ref):
    dinv = dinv_ref[...]
    h = jax.nn.relu(dinv * (aA_ref[...] + aB_ref[...] + y_ref[...]) + b_ref[...])
    yn_ref[...] = dinv * jnp.dot(
        h, w_ref[...], preferred_element_type=jnp.float32
    )


def _tc2_body(
    dinv_ref, ya_ref, yb_ref, aAa_ref, aBa_ref, aAb_ref, aBb_ref,
    ba_ref, bb_ref, wa_ref, wb_ref, yn_ref,
):
    dinv = dinv_ref[...]
    ha = jax.nn.relu(dinv * (aAa_ref[...] + aBa_ref[...] + ya_ref[...]) + ba_ref[...])
    hb = jax.nn.relu(dinv * (aAb_ref[...] + aBb_ref[...] + yb_ref[...]) + bb_ref[...])
    yn_ref[...] = dinv * (
        jnp.dot(ha, wa_ref[...], preferred_element_type=jnp.float32)
        + jnp.dot(hb, wb_ref[...], preferred_element_type=jnp.float32)
    )


def _tc3_body(dinv_ref, y_ref, aA_ref, aB_ref, b_ref, h_ref):
    dinv = dinv_ref[...]
    h_ref[...] = jax.nn.relu(
        dinv * (aA_ref[...] + aB_ref[...] + y_ref[...]) + b_ref[...]
    )


def _head_body(h_ref, fcW_ref, fcb_ref, fc2W_ref, fc2b_ref, o_ref):
    t = jnp.dot(h_ref[...], fcW_ref[...], preferred_element_type=jnp.float32)
    t = t + fcb_ref[...]
    o_ref[...] = (
        jnp.dot(t, fc2W_ref[...], preferred_element_type=jnp.float32) + fc2b_ref[...]
    )


def _head(hr, fcW, fcb, fc2W, fc2b):
    BG = 512
    return pl.pallas_call(
        _head_body,
        grid=(pl.cdiv(G, BG),),
        in_specs=[
            pl.BlockSpec((BG, 360), lambda i: (i, 0)),
            pl.BlockSpec((360, 120), lambda i: (0, 0)),
            pl.BlockSpec((120,), lambda i: (0,)),
            pl.BlockSpec((120, 36), lambda i: (0, 0)),
            pl.BlockSpec((36,), lambda i: (0,)),
        ],
        out_specs=pl.BlockSpec((BG, 36), lambda i: (i, 0)),
        out_shape=jax.ShapeDtypeStruct((G, 36), jnp.float32),
    )(hr, fcW, fcb, fc2W, fc2b)


def _pad2(w):
    return jnp.pad(w, ((0, D - w.shape[0]), (0, D - w.shape[1])))


def _kron8(w):
    return jnp.kron(jnp.eye(8, dtype=jnp.float32), _pad2(w))


def _bt(b):
    return jnp.tile(jnp.pad(b, (0, D - b.shape[0])), 8)


def kernel(x, edge_index, W1, b1, W2, b2, W3, b3, fcW, fcb, fc2W, fc2b):
    zeros2 = jnp.zeros((N_PAD, D), jnp.float32)
    ones2 = jnp.ones((SUP, D), jnp.float32)
    src3 = edge_index[0].reshape(S_TOT, K, CH)
    dst3 = edge_index[1].reshape(S_TOT, K, CH)
    xv = jnp.pad(x, ((0, N_PAD - N), (0, D - 6))).reshape(RV, 128)
    W1k = _kron8(W1)
    W2ak = _kron8(W2[:, :12])
    W2bk = _kron8(W2[:, 12:])
    W3ak = _kron8(W3[:12, :])
    W3bk = _kron8(W3[12:, :])
    b1t = _bt(b1)
    b2at = _bt(b2[:12])
    b2bt = _bt(b2[12:])
    b3t = _bt(b3)

    deg = _sc_deg(dst3, ones2, zeros2).reshape(2, RV, 128)
    dinv, y1 = _node_call(_tc0_body, 2, xv, deg[0], deg[1], W1k)

    a1 = _sc_agg(src3, dst3, y1.reshape(N_PAD, D), zeros2).reshape(2, RV, 128)
    y2a = _node_call(_tc_mid_body, 1, dinv, y1, a1[0], a1[1], b1t, W2ak)
    y2b = _node_call(_tc_mid_body, 1, dinv, y1, a1[0], a1[1], b1t, W2bk)

    a2a = _sc_agg(src3, dst3, y2a.reshape(N_PAD, D), zeros2).reshape(2, RV, 128)
    a2b = _sc_agg(src3, dst3, y2b.reshape(N_PAD, D), zeros2).reshape(2, RV, 128)
    y3 = _node_call(
        _tc2_body, 1, dinv, y2a, y2b, a2a[0], a2a[1], a2b[0], a2b[1],
        b2at, b2bt, W3ak, W3bk,
    )

    a3 = _sc_agg(src3, dst3, y3.reshape(N_PAD, D), zeros2).reshape(2, RV, 128)
    h3 = _node_call(_tc3_body, 1, dinv, y3, a3[0], a3[1], b3t)

    hr = h3.reshape(N_PAD, D)[:N, :12].reshape(G, 360)
    return _head(hr, fcW, fcb, fc2W, fc2b)


# speedup vs baseline: 68.3227x; 1.0848x over previous
import functools

import jax
import jax.numpy as jnp
from jax import lax
from jax.experimental import pallas as pl
from jax.experimental.pallas import tpu as pltpu
from jax.experimental.pallas import tpu_sc as plsc

N = 99990
E = 6399360
G = N // 30
D = 16
CH = 128
K = 5
SUP = K * CH
S_TOT = E // SUP
NC = 2
NS = 16
NW = NC * NS
ROWS = 6336
N_PAD = NS * ROWS
RV = N_PAD * D // 128

_MESH = plsc.VectorSubcoreMesh(core_axis_name="c", subcore_axis_name="s")
_SC_PARAMS = pltpu.CompilerParams(use_tc_tiling_on_sc=False)


def _tile_super_range(c, s):
    w = c * NS + s
    return (w * S_TOT) // NW, ((w + 1) * S_TOT) // NW


def _zero_acc(s, zeros_hbm, acc):
    lo = s * ROWS
    pltpu.sync_copy(zeros_hbm.at[pl.ds(lo, ROWS)], acc.at[pl.ds(lo, ROWS)])


def _writeback(c, s, acc, out_hbm):
    lo = s * ROWS
    pltpu.sync_copy(acc.at[pl.ds(lo, ROWS)], out_hbm.at[c, pl.ds(lo, ROWS)])


def _agg_body(src_hbm, dst_hbm, y_hbm, zeros_hbm, out_hbm,
              si, dis, rows, acc, isem, gsem, ssem):
    c = lax.axis_index("c")
    s = lax.axis_index("s")
    _zero_acc(s, zeros_hbm, acc)
    plsc.subcore_barrier()
    t_lo, t_hi = _tile_super_range(c, s)

    def _process(sup, sib, disb):
        gs = [
            pltpu.async_copy(
                y_hbm.at[sib.at[j]], rows.at[pl.ds(j * CH, CH)], gsem
            )
            for j in range(K)
        ]
        ss = []
        for j in range(K):
            gs[j].wait()
            ss.append(
                pltpu.async_copy(
                    rows.at[pl.ds(j * CH, CH)], acc.at[disb[j]], ssem, add=True
                )
            )
        for t in ss:
            t.wait()

    @pl.loop(t_lo, t_hi, step=2)
    def _(sup):
        ia = [pltpu.async_copy(src_hbm.at[sup], si[0], isem)] + [
            pltpu.async_copy(dst_hbm.at[sup, j], dis[j], isem) for j in range(K)
        ]
        have_b = sup + 1 < t_hi

        @pl.when(have_b)
        def _():
            ib = [pltpu.async_copy(src_hbm.at[sup + 1], si[1], isem)] + [
                pltpu.async_copy(dst_hbm.at[sup + 1, j], dis[K + j], isem)
                for j in range(K)
            ]
            for i1 in ib:
                i1.wait()

        for i1 in ia:
            i1.wait()
        _process(sup, si[0], dis[:K])

        @pl.when(have_b)
        def _():
            _process(sup + 1, si[1], dis[K:])

    plsc.subcore_barrier()
    _writeback(c, s, acc, out_hbm)


@functools.partial(
    pl.kernel,
    out_type=jax.ShapeDtypeStruct((NC, N_PAD, D), jnp.float32),
    mesh=_MESH,
    scratch_types=[
        [pltpu.VMEM((K, CH), jnp.int32) for _ in range(2)],
        [pltpu.VMEM((CH,), jnp.int32) for _ in range(2 * K)],
        pltpu.VMEM((SUP, D), jnp.float32),
        pltpu.VMEM_SHARED((N_PAD, D), jnp.float32),
        pltpu.SemaphoreType.DMA,
        pltpu.SemaphoreType.DMA,
        pltpu.SemaphoreType.DMA,
    ],
    compiler_params=_SC_PARAMS,
)
def _sc_agg(src_hbm, dst_hbm, y_hbm, zeros_hbm, out_hbm,
            si, dis, rows, acc, isem, gsem, ssem):
    _agg_body(src_hbm, dst_hbm, y_hbm, zeros_hbm, out_hbm,
              si, dis, rows, acc, isem, gsem, ssem)


def _deg_body(dst_hbm, ones_hbm, zeros_hbm, out_hbm, dis, ones, acc, isem, ssem):
    c = lax.axis_index("c")
    s = lax.axis_index("s")
    _zero_acc(s, zeros_hbm, acc)
    pltpu.sync_copy(ones_hbm, ones)
    plsc.subcore_barrier()
    t_lo, t_hi = _tile_super_range(c, s)

    @pl.loop(t_lo, t_hi)
    def _(sup):
        ii = [
            pltpu.async_copy(dst_hbm.at[sup, j], dis[j], isem) for j in range(K)
        ]
        for i1 in ii:
            i1.wait()
        ss = [
            pltpu.async_copy(
                ones.at[pl.ds(j * CH, CH)], acc.at[dis[j]], ssem, add=True
            )
            for j in range(K)
        ]
        for t in ss:
            t.wait()

    plsc.subcore_barrier()
    _writeback(c, s, acc, out_hbm)


@functools.partial(
    pl.kernel,
    out_type=jax.ShapeDtypeStruct((NC, N_PAD, D), jnp.float32),
    mesh=_MESH,
    scratch_types=[
        [pltpu.VMEM((CH,), jnp.int32) for _ in range(K)],
        pltpu.VMEM((SUP, D), jnp.float32),
        pltpu.VMEM_SHARED((N_PAD, D), jnp.float32),
        pltpu.SemaphoreType.DMA,
        pltpu.SemaphoreType.DMA,
    ],
    compiler_params=_SC_PARAMS,
)
def _sc_deg(dst_hbm, ones_hbm, zeros_hbm, out_hbm, dis, ones, acc, isem, ssem):
    _deg_body(dst_hbm, ones_hbm, zeros_hbm, out_hbm, dis, ones, acc, isem, ssem)


_BV = 2112


def _node_call(body, n_out, *args):
    grid = (RV // _BV,)
    spec = pl.BlockSpec((_BV, 128), lambda i: (i, 0))

    def mk_spec(a):
        if a.ndim == 2 and a.shape[0] == RV:
            return pl.BlockSpec((_BV, 128), lambda i: (i, 0))
        return pl.BlockSpec(a.shape, lambda i: (0,) * a.ndim)

    return pl.pallas_call(
        body,
        grid=grid,
        in_specs=[mk_spec(a) for a in args],
        out_specs=[spec] * n_out if n_out > 1 else spec,
        out_shape=(
            [jax.ShapeDtypeStruct((RV, 128), jnp.float32)] * n_out
            if n_out > 1
            else jax.ShapeDtypeStruct((RV, 128), jnp.float32)
        ),
    )(*args)


def _tc0_body(x_ref, dA_ref, dB_ref, w1_ref, dinv_ref, y1_ref):
    deg = dA_ref[...] + dB_ref[...] + 1.0
    dinv = lax.rsqrt(deg)
    dinv_ref[...] = dinv
    y1_ref[...] = dinv * jnp.dot(
        x_ref[...], w1_ref[...], preferred_element_type=jnp.float32
    )


def _tc_mid_body(dinv_ref, y_ref, aA_ref, aB_ref, b_ref, wa_ref, wb_ref,
                 ya_ref, yb_ref):
    dinv = dinv_ref[...]
    h = jax.nn.relu(dinv * (aA_ref[...] + aB_ref[...] + y_ref[...]) + b_ref[...])
    ya_ref[...] = dinv * jnp.dot(h, wa_ref[...], preferred_element_type=jnp.float32)
    yb_ref[...] = dinv * jnp.dot(h, wb_ref[...], preferred_element_type=jnp.float32)


def _tc2_body(
    dinv_ref, ya_ref, yb_ref, aAa_ref, aBa_ref, aAb_ref, aBb_ref,
    ba_ref, bb_ref, wa_ref, wb_ref, yn_ref,
):
    dinv = dinv_ref[...]
    ha = jax.nn.relu(dinv * (aAa_ref[...] + aBa_ref[...] + ya_ref[...]) + ba_ref[...])
    hb = jax.nn.relu(dinv * (aAb_ref[...] + aBb_ref[...] + yb_ref[...]) + bb_ref[...])
    yn_ref[...] = dinv * (
        jnp.dot(ha, wa_ref[...], preferred_element_type=jnp.float32)
        + jnp.dot(hb, wb_ref[...], preferred_element_type=jnp.float32)
    )


def _tc3_body(dinv_ref, y_ref, aA_ref, aB_ref, b_ref, h_ref):
    dinv = dinv_ref[...]
    h_ref[...] = jax.nn.relu(
        dinv * (aA_ref[...] + aB_ref[...] + y_ref[...]) + b_ref[...]
    )


def _head_body(h_ref, fcW_ref, fcb_ref, fc2W_ref, fc2b_ref, o_ref):
    t = jnp.dot(h_ref[...], fcW_ref[...], preferred_element_type=jnp.float32)
    t = t + fcb_ref[...]
    o_ref[...] = (
        jnp.dot(t, fc2W_ref[...], preferred_element_type=jnp.float32) + fc2b_ref[...]
    )


def _head(hr, fcW, fcb, fc2W, fc2b):
    BG = 512
    return pl.pallas_call(
        _head_body,
        grid=(pl.cdiv(G, BG),),
        in_specs=[
            pl.BlockSpec((BG, 360), lambda i: (i, 0)),
            pl.BlockSpec((360, 120), lambda i: (0, 0)),
            pl.BlockSpec((120,), lambda i: (0,)),
            pl.BlockSpec((120, 36), lambda i: (0, 0)),
            pl.BlockSpec((36,), lambda i: (0,)),
        ],
        out_specs=pl.BlockSpec((BG, 36), lambda i: (i, 0)),
        out_shape=jax.ShapeDtypeStruct((G, 36), jnp.float32),
    )(hr, fcW, fcb, fc2W, fc2b)


def _pad2(w):
    return jnp.pad(w, ((0, D - w.shape[0]), (0, D - w.shape[1])))


def _kron8(w):
    return jnp.kron(jnp.eye(8, dtype=jnp.float32), _pad2(w))


def _bt(b):
    return jnp.tile(jnp.pad(b, (0, D - b.shape[0])), 8)


def kernel(x, edge_index, W1, b1, W2, b2, W3, b3, fcW, fcb, fc2W, fc2b):
    zeros2 = jnp.zeros((N_PAD, D), jnp.float32)
    ones2 = jnp.ones((SUP, D), jnp.float32)
    src3 = edge_index[0].reshape(S_TOT, K, CH)
    dst3 = edge_index[1].reshape(S_TOT, K, CH)
    xv = jnp.pad(x, ((0, N_PAD - N), (0, D - 6))).reshape(RV, 128)
    W1k = _kron8(W1)
    W2ak = _kron8(W2[:, :12])
    W2bk = _kron8(W2[:, 12:])
    W3ak = _kron8(W3[:12, :])
    W3bk = _kron8(W3[12:, :])
    b1t = _bt(b1)
    b2at = _bt(b2[:12])
    b2bt = _bt(b2[12:])
    b3t = _bt(b3)

    deg = _sc_deg(dst3, ones2, zeros2).reshape(2, RV, 128)
    dinv, y1 = _node_call(_tc0_body, 2, xv, deg[0], deg[1], W1k)

    a1 = _sc_agg(src3, dst3, y1.reshape(N_PAD, D), zeros2).reshape(2, RV, 128)
    y2a, y2b = _node_call(_tc_mid_body, 2, dinv, y1, a1[0], a1[1], b1t, W2ak, W2bk)

    a2a = _sc_agg(src3, dst3, y2a.reshape(N_PAD, D), zeros2).reshape(2, RV, 128)
    a2b = _sc_agg(src3, dst3, y2b.reshape(N_PAD, D), zeros2).reshape(2, RV, 128)
    y3 = _node_call(
        _tc2_body, 1, dinv, y2a, y2b, a2a[0], a2a[1], a2b[0], a2b[1],
        b2at, b2bt, W3ak, W3bk,
    )

    a3 = _sc_agg(src3, dst3, y3.reshape(N_PAD, D), zeros2).reshape(2, RV, 128)
    h3 = _node_call(_tc3_body, 1, dinv, y3, a3[0], a3[1], b3t)

    hr = h3.reshape(N_PAD, D)[:N, :12].reshape(G, 360)
    return _head(hr, fcW, fcb, fc2W, fc2b)
